# Initial kernel scaffold; baseline (speedup 1.0000x reference)
import jax
import jax.numpy as jnp
from jax.experimental import pallas as pl


def kernel(x_user, x_item, edge_index_u2i, edge_index_i2u, Ws0u, Ws0i, Wm0_u2i, Wm0_i2u, b0u, b0i, Ws1u, Ws1i, Wm1_u2i, Wm1_i2u, b1u, b1i):
    raise NotImplementedError("write your pallas kernel here")


# speedup vs baseline: 13.4638x; 13.4638x over previous
import functools

import jax
import jax.numpy as jnp
from jax import lax
from jax.experimental import pallas as pl
from jax.experimental.pallas import tpu as pltpu
from jax.experimental.pallas import tpu_sc as plsc

N = 10000
E = 256000
D = 128

NC = 2
NS = 16
CHUNK = 128
NBUF = 2
EPT = E // NS
NCH = EPT // CHUNK
DRC = 80
NDC = N // DRC


def _sc_body(ta, tb, sa, da, sb, db, zeros_hbm, out_a, out_b,
             acc, src_v, dst_ring, rows_v, gsem, dsem):
    c = lax.axis_index("c")
    s = lax.axis_index("s")

    def do_relation(t_hbm, src_hbm, dst_hbm, out_hbm):
        pltpu.sync_copy(src_hbm.at[pl.ds(s * EPT, EPT)], src_v)
        zb = rows_v.at[0, pl.ds(0, DRC)]
        pltpu.sync_copy(zeros_hbm, zb)
        for m in range(8):
            j = m * NS + s

            @pl.when(j < NDC)
            def _():
                pltpu.sync_copy(zb, acc.at[pl.ds(j * DRC, DRC)])
        plsc.subcore_barrier()

        def issue(jn, b):
            pltpu.async_copy(
                dst_hbm.at[pl.ds(s * EPT + jn * CHUNK, CHUNK)],
                dst_ring.at[b], dsem.at[b])
            pltpu.async_copy(
                t_hbm.at[src_v.at[pl.ds(jn * CHUNK, CHUNK)]],
                rows_v.at[b], gsem.at[b])

        def step(j, b):
            pltpu.make_async_copy(
                dst_hbm.at[pl.ds(0, CHUNK)], dst_ring.at[b],
                dsem.at[b]).wait()
            pltpu.make_async_copy(
                t_hbm.at[pl.ds(0, CHUNK)], rows_v.at[b], gsem.at[b]).wait()
            pltpu.sync_copy(rows_v.at[b], acc.at[dst_ring.at[b]], add=True)

            @pl.when(j + NBUF < NCH)
            def _():
                issue(j + NBUF, b)

        for b in range(NBUF):
            issue(b, b)

        def outer(j0, carry):
            for b in range(NBUF):
                step(j0 * NBUF + b, b)
            return carry

        lax.fori_loop(0, NCH // NBUF, outer, 0)
        step(NCH - 1, 0)
        plsc.subcore_barrier()

        for m in range(8):
            j = m * NS + s

            @pl.when(j < NDC)
            def _():
                base = j * DRC
                pltpu.sync_copy(acc.at[pl.ds(base, DRC)], zb)
                pltpu.sync_copy(zb, out_hbm.at[pl.ds(base, DRC)])

    @pl.when(c == 0)
    def _():
        do_relation(ta, sa, da, out_a)

    @pl.when(c == 1)
    def _():
        do_relation(tb, sb, db, out_b)


def _sc_agg(t_a, t_b, src_a, dst_a, src_b, dst_b, zeros):
    mesh = plsc.VectorSubcoreMesh(
        core_axis_name="c", subcore_axis_name="s",
        num_cores=NC, num_subcores=NS)
    f = pl.kernel(
        _sc_body,
        out_type=[jax.ShapeDtypeStruct((N, D), jnp.float32),
                  jax.ShapeDtypeStruct((N, D), jnp.float32)],
        mesh=mesh,
        scratch_types=[
            pltpu.VMEM_SHARED((N, D), jnp.float32),
            pltpu.VMEM((EPT,), jnp.int32),
            pltpu.VMEM((8, CHUNK), jnp.int32),
            pltpu.VMEM((NBUF, CHUNK, D), jnp.float32),
            pltpu.SemaphoreType.DMA((NBUF,)),
            pltpu.SemaphoreType.DMA((NBUF,)),
        ],
    )
    return f(t_a, t_b, src_a, dst_a, src_b, dst_b, zeros)


_BR = 2000


def _mm2_body(x, wa, wb, oa, ob):
    oa[...] = jnp.dot(x[...], wa[...], preferred_element_type=jnp.float32)
    ob[...] = jnp.dot(x[...], wb[...], preferred_element_type=jnp.float32)


def _mm2(x, wa, wb):
    grid = (N // _BR,)
    bs_x = pl.BlockSpec((_BR, D), lambda i: (i, 0))
    bs_w = pl.BlockSpec((D, D), lambda i: (0, 0))
    return pl.pallas_call(
        _mm2_body,
        grid=grid,
        in_specs=[bs_x, bs_w, bs_w],
        out_specs=[bs_x, bs_x],
        out_shape=[jax.ShapeDtypeStruct((N, D), jnp.float32)] * 2,
    )(x, wa, wb)


def _post_mm2_body(sref, aref, bref, wa, wb, oh, oa, ob):
    h = jnp.maximum(sref[...] + aref[...] + bref[...], 0.0)
    oh[...] = h
    oa[...] = jnp.dot(h, wa[...], preferred_element_type=jnp.float32)
    ob[...] = jnp.dot(h, wb[...], preferred_element_type=jnp.float32)


def _post_mm2(sx, agg, bias, wa, wb):
    grid = (N // _BR,)
    bs_x = pl.BlockSpec((_BR, D), lambda i: (i, 0))
    bs_b = pl.BlockSpec((1, D), lambda i: (0, 0))
    bs_w = pl.BlockSpec((D, D), lambda i: (0, 0))
    return pl.pallas_call(
        _post_mm2_body,
        grid=grid,
        in_specs=[bs_x, bs_x, bs_b, bs_w, bs_w],
        out_specs=[bs_x, bs_x, bs_x],
        out_shape=[jax.ShapeDtypeStruct((N, D), jnp.float32)] * 3,
    )(sx, agg, bias.reshape(1, D), wa, wb)


def _post_body(sref, aref, bref, oh):
    oh[...] = jnp.maximum(sref[...] + aref[...] + bref[...], 0.0)


def _post(sx, agg, bias):
    grid = (N // _BR,)
    bs_x = pl.BlockSpec((_BR, D), lambda i: (i, 0))
    bs_b = pl.BlockSpec((1, D), lambda i: (0, 0))
    return pl.pallas_call(
        _post_body,
        grid=grid,
        in_specs=[bs_x, bs_x, bs_b],
        out_specs=bs_x,
        out_shape=jax.ShapeDtypeStruct((N, D), jnp.float32),
    )(sx, agg, bias.reshape(1, D))


def kernel(x_user, x_item, edge_index_u2i, edge_index_i2u,
           Ws0u, Ws0i, Wm0_u2i, Wm0_i2u, b0u, b0i,
           Ws1u, Ws1i, Wm1_u2i, Wm1_i2u, b1u, b1i):
    src_a = edge_index_u2i[0]
    dst_a = edge_index_u2i[1]
    src_b = edge_index_i2u[0]
    dst_b = edge_index_i2u[1]
    zeros = jnp.zeros((DRC, D), jnp.float32)

    s0u, t0u = _mm2(x_user, Ws0u, Wm0_u2i)
    s0i, t0i = _mm2(x_item, Ws0i, Wm0_i2u)
    agg_i0, agg_u0 = _sc_agg(t0u, t0i, src_a, dst_a, src_b, dst_b, zeros)
    h0u, s1u, t1u = _post_mm2(s0u, agg_u0, b0u, Ws1u, Wm1_u2i)
    h0i, s1i, t1i = _post_mm2(s0i, agg_i0, b0i, Ws1i, Wm1_i2u)
    agg_i1, agg_u1 = _sc_agg(t1u, t1i, src_a, dst_a, src_b, dst_b, zeros)
    h1u = _post(s1u, agg_u1, b1u)
    h1i = _post(s1i, agg_i1, b1i)
    return (h0u, h0i, h1u, h1i)
